# baseline (speedup 1.0000x reference)
import functools

import jax
import jax.numpy as jnp
from jax import lax
from jax.experimental import pallas as pl
from jax.experimental.pallas import tpu as pltpu
from jax.experimental.pallas import tpu_sc as plsc

NUM_WORDS = 1000000
NUM_LABELS = 100
EMBED_DIM = 32
HIDDEN_DIM = 64
BATCH = 16384

_CW = 2048
_NBLK = pl.cdiv(NUM_WORDS, _CW)
_T2_ROWS = _NBLK * 512


def _relayout_body(t_ref, o_ref):
    parts = [t_ref[:, 512 * a:512 * (a + 1)] for a in range(4)]
    v = jnp.concatenate(parts, axis=0)
    eye = jnp.eye(512, dtype=jnp.bfloat16)
    o_ref[...] = lax.dot_general(
        eye, v.astype(jnp.bfloat16), (((1,), (1,)), ((), ())),
        preferred_element_type=jnp.float32)


def _relayout(tableT):
    return pl.pallas_call(
        _relayout_body,
        grid=(_NBLK,),
        in_specs=[pl.BlockSpec((EMBED_DIM, _CW), lambda i: (0, i))],
        out_specs=pl.BlockSpec((512, 128), lambda i: (i, 0)),
        out_shape=jax.ShapeDtypeStruct((_T2_ROWS, 128), jnp.float32),
    )(tableT)


def _make_gather(B: int, D: int):
    info = plsc.get_sparse_core_info()
    NC, NS = info.num_cores, info.num_subcores
    NW = NC * NS
    assert B % (8 * NW) == 0
    b_per_w = B // NW
    mesh = plsc.VectorSubcoreMesh(core_axis_name="c", subcore_axis_name="s")

    @functools.partial(
        pl.kernel,
        mesh=mesh,
        out_type=jax.ShapeDtypeStruct((B, D), jnp.float32),
        scratch_types=[
            pltpu.VMEM((b_per_w,), jnp.int32),
            pltpu.VMEM((b_per_w,), jnp.int32),
            pltpu.VMEM((b_per_w, D), jnp.float32),
            pltpu.SemaphoreType.DMA,
        ],
        compiler_params=pltpu.CompilerParams(use_tc_tiling_on_sc=False),
    )
    def gather_k(idx_hbm, table_hbm, out_hbm, idx_v, idx2_v, rows_v, sem):
        wid = lax.axis_index("s") * NC + lax.axis_index("c")
        base = wid * b_per_w
        pltpu.sync_copy(idx_hbm.at[pl.ds(base, b_per_w)], idx_v)

        def body(j, _):
            w = idx_v[pl.ds(j * 16, 16)]
            i = lax.shift_right_logical(w, 11)
            q = lax.bitwise_and(w, 2047)
            a = lax.shift_right_logical(q, 9)
            r = lax.bitwise_and(q, 511)
            s = lax.bitwise_or(
                lax.shift_left(i * 512 + r, 2), a)
            idx2_v[pl.ds(j * 16, 16)] = s
            return 0

        lax.fori_loop(0, b_per_w // 16, body, 0, unroll=4)

        pltpu.async_copy(table_hbm.at[idx2_v], rows_v, sem).wait()
        pltpu.sync_copy(rows_v, out_hbm.at[pl.ds(base, b_per_w)])

    return gather_k


def _mlp_body(e_ref, w1_ref, b1_ref, w2_ref, b2_ref, o_ref):
    e = e_ref[...]
    h = lax.dot_general(e, w1_ref[...], (((1,), (1,)), ((), ())),
                        preferred_element_type=jnp.float32)
    h = jnp.maximum(h + b1_ref[...], 0.0)
    o = lax.dot_general(h, w2_ref[...], (((1,), (1,)), ((), ())),
                        preferred_element_type=jnp.float32)
    o_ref[...] = o + b2_ref[...]


def _mlp(emb, W1, b1, W2, b2):
    B = emb.shape[0]
    BB = 2048
    grid = (B // BB,)
    return pl.pallas_call(
        _mlp_body,
        grid=grid,
        in_specs=[
            pl.BlockSpec((BB, EMBED_DIM), lambda i: (i, 0)),
            pl.BlockSpec((HIDDEN_DIM, EMBED_DIM), lambda i: (0, 0)),
            pl.BlockSpec((1, HIDDEN_DIM), lambda i: (0, 0)),
            pl.BlockSpec((NUM_LABELS, HIDDEN_DIM), lambda i: (0, 0)),
            pl.BlockSpec((1, NUM_LABELS), lambda i: (0, 0)),
        ],
        out_specs=pl.BlockSpec((BB, NUM_LABELS), lambda i: (i, 0)),
        out_shape=jax.ShapeDtypeStruct((B, NUM_LABELS), jnp.float32),
    )(emb, W1, b1.reshape(1, -1), W2, b2.reshape(1, -1))


def kernel(x, table, W1, b1, W2, b2):
    t2 = _relayout(table.T)
    table_lin = t2.reshape(_T2_ROWS * 4, EMBED_DIM)
    gather_k = _make_gather(BATCH, EMBED_DIM)
    emb = gather_k(x.astype(jnp.int32), table_lin)
    return _mlp(emb, W1, b1, W2, b2)


# speedup vs baseline: 2.1758x; 2.1758x over previous
import functools

import jax
import jax.numpy as jnp
from jax import lax
from jax.experimental import pallas as pl
from jax.experimental.pallas import tpu as pltpu
from jax.experimental.pallas import tpu_sc as plsc

NUM_WORDS = 1000000
NUM_LABELS = 100
EMBED_DIM = 32
HIDDEN_DIM = 64
BATCH = 16384

_CW = 8192
_Q = _CW // 4
_SH_CW = _CW.bit_length() - 1
_SH_Q = _Q.bit_length() - 1
_NBLK = pl.cdiv(NUM_WORDS, _CW)
_T2_ROWS = _NBLK * _Q


def _relayout_body(t_ref, o_ref):
    q = _CW // 4
    parts = [t_ref[:, q * a:q * (a + 1)] for a in range(4)]
    v = jnp.concatenate(parts, axis=0)
    o_ref[...] = v.T


def _relayout(tableT):
    return pl.pallas_call(
        _relayout_body,
        grid=(_NBLK,),
        in_specs=[pl.BlockSpec((EMBED_DIM, _CW), lambda i: (0, i))],
        out_specs=pl.BlockSpec((_Q, 128), lambda i: (i, 0)),
        out_shape=jax.ShapeDtypeStruct((_T2_ROWS, 128), jnp.float32),
    )(tableT)


def _make_gather(B: int, D: int):
    info = plsc.get_sparse_core_info()
    NC, NS = info.num_cores, info.num_subcores
    NW = NC * NS
    assert B % (8 * NW) == 0
    b_per_w = B // NW
    mesh = plsc.VectorSubcoreMesh(core_axis_name="c", subcore_axis_name="s")

    @functools.partial(
        pl.kernel,
        mesh=mesh,
        out_type=jax.ShapeDtypeStruct((B, D), jnp.float32),
        scratch_types=[
            pltpu.VMEM((b_per_w,), jnp.int32),
            pltpu.VMEM((b_per_w,), jnp.int32),
            pltpu.VMEM((b_per_w, D), jnp.float32),
            pltpu.SemaphoreType.DMA,
        ],
        compiler_params=pltpu.CompilerParams(use_tc_tiling_on_sc=False),
    )
    def gather_k(idx_hbm, table_hbm, out_hbm, idx_v, idx2_v, rows_v, sem):
        wid = lax.axis_index("s") * NC + lax.axis_index("c")
        base = wid * b_per_w
        pltpu.sync_copy(idx_hbm.at[pl.ds(base, b_per_w)], idx_v)

        def body(j, _):
            w = idx_v[pl.ds(j * 16, 16)]
            i = lax.shift_right_logical(w, _SH_CW)
            q = lax.bitwise_and(w, _CW - 1)
            a = lax.shift_right_logical(q, _SH_Q)
            r = lax.bitwise_and(q, _Q - 1)
            s = lax.bitwise_or(
                lax.shift_left(i * _Q + r, 2), a)
            idx2_v[pl.ds(j * 16, 16)] = s
            return 0

        lax.fori_loop(0, b_per_w // 16, body, 0, unroll=4)

        pltpu.async_copy(table_hbm.at[idx2_v], rows_v, sem).wait()
        pltpu.sync_copy(rows_v, out_hbm.at[pl.ds(base, b_per_w)])

    return gather_k


def _mlp_body(e_ref, w1_ref, b1_ref, w2_ref, b2_ref, o_ref):
    e = e_ref[...]
    h = lax.dot_general(e, w1_ref[...], (((1,), (1,)), ((), ())),
                        preferred_element_type=jnp.float32)
    h = jnp.maximum(h + b1_ref[...], 0.0)
    o = lax.dot_general(h, w2_ref[...], (((1,), (1,)), ((), ())),
                        preferred_element_type=jnp.float32)
    o_ref[...] = o + b2_ref[...]


def _mlp(emb, W1, b1, W2, b2):
    B = emb.shape[0]
    BB = 2048
    grid = (B // BB,)
    return pl.pallas_call(
        _mlp_body,
        grid=grid,
        in_specs=[
            pl.BlockSpec((BB, EMBED_DIM), lambda i: (i, 0)),
            pl.BlockSpec((HIDDEN_DIM, EMBED_DIM), lambda i: (0, 0)),
            pl.BlockSpec((1, HIDDEN_DIM), lambda i: (0, 0)),
            pl.BlockSpec((NUM_LABELS, HIDDEN_DIM), lambda i: (0, 0)),
            pl.BlockSpec((1, NUM_LABELS), lambda i: (0, 0)),
        ],
        out_specs=pl.BlockSpec((BB, NUM_LABELS), lambda i: (i, 0)),
        out_shape=jax.ShapeDtypeStruct((B, NUM_LABELS), jnp.float32),
    )(emb, W1, b1.reshape(1, -1), W2, b2.reshape(1, -1))


def kernel(x, table, W1, b1, W2, b2):
    t2 = _relayout(table.T)
    table_lin = t2.reshape(_T2_ROWS * 4, EMBED_DIM)
    gather_k = _make_gather(BATCH, EMBED_DIM)
    emb = gather_k(x.astype(jnp.int32), table_lin)
    return _mlp(emb, W1, b1, W2, b2)


# speedup vs baseline: 3.0463x; 1.4001x over previous
import functools

import jax
import jax.numpy as jnp
from jax import lax
from jax.experimental import pallas as pl
from jax.experimental.pallas import tpu as pltpu
from jax.experimental.pallas import tpu_sc as plsc

NUM_WORDS = 1000000
NUM_LABELS = 100
EMBED_DIM = 32
HIDDEN_DIM = 64
BATCH = 16384

_CW = 32768
_Q = _CW // 4
_SH_CW = _CW.bit_length() - 1
_SH_Q = _Q.bit_length() - 1
_NBLK = pl.cdiv(NUM_WORDS, _CW)
_T2_ROWS = _NBLK * _Q


def _relayout_body(t_ref, o_ref):
    q = _CW // 4
    parts = [t_ref[:, q * a:q * (a + 1)] for a in range(4)]
    v = jnp.concatenate(parts, axis=0)
    o_ref[...] = v.T


def _relayout(tableT):
    return pl.pallas_call(
        _relayout_body,
        grid=(_NBLK,),
        in_specs=[pl.BlockSpec((EMBED_DIM, _CW), lambda i: (0, i))],
        out_specs=pl.BlockSpec((_Q, 128), lambda i: (i, 0)),
        out_shape=jax.ShapeDtypeStruct((_T2_ROWS, 128), jnp.float32),
    )(tableT)


def _make_gather(B: int, D: int):
    info = plsc.get_sparse_core_info()
    NC, NS = info.num_cores, info.num_subcores
    NW = NC * NS
    assert B % (8 * NW) == 0
    b_per_w = B // NW
    mesh = plsc.VectorSubcoreMesh(core_axis_name="c", subcore_axis_name="s")

    @functools.partial(
        pl.kernel,
        mesh=mesh,
        out_type=jax.ShapeDtypeStruct((B, D), jnp.float32),
        scratch_types=[
            pltpu.VMEM((b_per_w,), jnp.int32),
            pltpu.VMEM((b_per_w,), jnp.int32),
            pltpu.VMEM((b_per_w, D), jnp.float32),
            pltpu.SemaphoreType.DMA,
        ],
        compiler_params=pltpu.CompilerParams(use_tc_tiling_on_sc=False),
    )
    def gather_k(idx_hbm, table_hbm, out_hbm, idx_v, idx2_v, rows_v, sem):
        wid = lax.axis_index("s") * NC + lax.axis_index("c")
        base = wid * b_per_w
        pltpu.sync_copy(idx_hbm.at[pl.ds(base, b_per_w)], idx_v)

        def body(j, _):
            w = idx_v[pl.ds(j * 16, 16)]
            i = lax.shift_right_logical(w, _SH_CW)
            q = lax.bitwise_and(w, _CW - 1)
            a = lax.shift_right_logical(q, _SH_Q)
            r = lax.bitwise_and(q, _Q - 1)
            s = lax.bitwise_or(
                lax.shift_left(i * _Q + r, 2), a)
            idx2_v[pl.ds(j * 16, 16)] = s
            return 0

        lax.fori_loop(0, b_per_w // 16, body, 0, unroll=4)

        pltpu.async_copy(table_hbm.at[idx2_v], rows_v, sem).wait()
        pltpu.sync_copy(rows_v, out_hbm.at[pl.ds(base, b_per_w)])

    return gather_k


def _mlp_body(e_ref, w1_ref, b1_ref, w2_ref, b2_ref, o_ref):
    e = e_ref[...]
    h = lax.dot_general(e, w1_ref[...], (((1,), (1,)), ((), ())),
                        preferred_element_type=jnp.float32)
    h = jnp.maximum(h + b1_ref[...], 0.0)
    o = lax.dot_general(h, w2_ref[...], (((1,), (1,)), ((), ())),
                        preferred_element_type=jnp.float32)
    o_ref[...] = o + b2_ref[...]


def _mlp(emb, W1, b1, W2, b2):
    B = emb.shape[0]
    BB = 2048
    grid = (B // BB,)
    return pl.pallas_call(
        _mlp_body,
        grid=grid,
        in_specs=[
            pl.BlockSpec((BB, EMBED_DIM), lambda i: (i, 0)),
            pl.BlockSpec((HIDDEN_DIM, EMBED_DIM), lambda i: (0, 0)),
            pl.BlockSpec((1, HIDDEN_DIM), lambda i: (0, 0)),
            pl.BlockSpec((NUM_LABELS, HIDDEN_DIM), lambda i: (0, 0)),
            pl.BlockSpec((1, NUM_LABELS), lambda i: (0, 0)),
        ],
        out_specs=pl.BlockSpec((BB, NUM_LABELS), lambda i: (i, 0)),
        out_shape=jax.ShapeDtypeStruct((B, NUM_LABELS), jnp.float32),
    )(emb, W1, b1.reshape(1, -1), W2, b2.reshape(1, -1))


def kernel(x, table, W1, b1, W2, b2):
    t2 = _relayout(table.T)
    table_lin = t2.reshape(_T2_ROWS * 4, EMBED_DIM)
    gather_k = _make_gather(BATCH, EMBED_DIM)
    emb = gather_k(x.astype(jnp.int32), table_lin)
    return _mlp(emb, W1, b1, W2, b2)


# speedup vs baseline: 3.0741x; 1.0091x over previous
import functools

import jax
import jax.numpy as jnp
from jax import lax
from jax.experimental import pallas as pl
from jax.experimental.pallas import tpu as pltpu
from jax.experimental.pallas import tpu_sc as plsc

NUM_WORDS = 1000000
NUM_LABELS = 100
EMBED_DIM = 32
HIDDEN_DIM = 64
BATCH = 16384

_CW = 65536
_Q = _CW // 4
_SH_CW = _CW.bit_length() - 1
_SH_Q = _Q.bit_length() - 1
_NBLK = pl.cdiv(NUM_WORDS, _CW)
_T2_ROWS = _NBLK * _Q


def _relayout_body(t_ref, o_ref):
    q = _CW // 4
    parts = [t_ref[:, q * a:q * (a + 1)] for a in range(4)]
    v = jnp.concatenate(parts, axis=0)
    o_ref[...] = v.T


def _relayout(tableT):
    return pl.pallas_call(
        _relayout_body,
        grid=(_NBLK,),
        in_specs=[pl.BlockSpec((EMBED_DIM, _CW), lambda i: (0, i))],
        out_specs=pl.BlockSpec((_Q, 128), lambda i: (i, 0)),
        out_shape=jax.ShapeDtypeStruct((_T2_ROWS, 128), jnp.float32),
    )(tableT)


def _make_gather(B: int, D: int):
    info = plsc.get_sparse_core_info()
    NC, NS = info.num_cores, info.num_subcores
    NW = NC * NS
    assert B % (8 * NW) == 0
    b_per_w = B // NW
    mesh = plsc.VectorSubcoreMesh(core_axis_name="c", subcore_axis_name="s")

    @functools.partial(
        pl.kernel,
        mesh=mesh,
        out_type=jax.ShapeDtypeStruct((B, D), jnp.float32),
        scratch_types=[
            pltpu.VMEM((b_per_w,), jnp.int32),
            pltpu.VMEM((b_per_w,), jnp.int32),
            pltpu.VMEM((b_per_w, D), jnp.float32),
            pltpu.SemaphoreType.DMA,
        ],
        compiler_params=pltpu.CompilerParams(use_tc_tiling_on_sc=False),
    )
    def gather_k(idx_hbm, table_hbm, out_hbm, idx_v, idx2_v, rows_v, sem):
        wid = lax.axis_index("s") * NC + lax.axis_index("c")
        base = wid * b_per_w
        pltpu.sync_copy(idx_hbm.at[pl.ds(base, b_per_w)], idx_v)

        def body(j, _):
            w = idx_v[pl.ds(j * 16, 16)]
            i = lax.shift_right_logical(w, _SH_CW)
            q = lax.bitwise_and(w, _CW - 1)
            a = lax.shift_right_logical(q, _SH_Q)
            r = lax.bitwise_and(q, _Q - 1)
            s = lax.bitwise_or(
                lax.shift_left(i * _Q + r, 2), a)
            idx2_v[pl.ds(j * 16, 16)] = s
            return 0

        lax.fori_loop(0, b_per_w // 16, body, 0, unroll=4)

        pltpu.async_copy(table_hbm.at[idx2_v], rows_v, sem).wait()
        pltpu.sync_copy(rows_v, out_hbm.at[pl.ds(base, b_per_w)])

    return gather_k


def _mlp_body(e_ref, w1_ref, b1_ref, w2_ref, b2_ref, o_ref):
    e = e_ref[...]
    h = lax.dot_general(e, w1_ref[...], (((1,), (1,)), ((), ())),
                        preferred_element_type=jnp.float32)
    h = jnp.maximum(h + b1_ref[...], 0.0)
    o = lax.dot_general(h, w2_ref[...], (((1,), (1,)), ((), ())),
                        preferred_element_type=jnp.float32)
    o_ref[...] = o + b2_ref[...]


def _mlp(emb, W1, b1, W2, b2):
    B = emb.shape[0]
    BB = 2048
    grid = (B // BB,)
    return pl.pallas_call(
        _mlp_body,
        grid=grid,
        in_specs=[
            pl.BlockSpec((BB, EMBED_DIM), lambda i: (i, 0)),
            pl.BlockSpec((HIDDEN_DIM, EMBED_DIM), lambda i: (0, 0)),
            pl.BlockSpec((1, HIDDEN_DIM), lambda i: (0, 0)),
            pl.BlockSpec((NUM_LABELS, HIDDEN_DIM), lambda i: (0, 0)),
            pl.BlockSpec((1, NUM_LABELS), lambda i: (0, 0)),
        ],
        out_specs=pl.BlockSpec((BB, NUM_LABELS), lambda i: (i, 0)),
        out_shape=jax.ShapeDtypeStruct((B, NUM_LABELS), jnp.float32),
    )(emb, W1, b1.reshape(1, -1), W2, b2.reshape(1, -1))


def kernel(x, table, W1, b1, W2, b2):
    t2 = _relayout(table.T)
    table_lin = t2.reshape(_T2_ROWS * 4, EMBED_DIM)
    gather_k = _make_gather(BATCH, EMBED_DIM)
    emb = gather_k(x.astype(jnp.int32), table_lin)
    return _mlp(emb, W1, b1, W2, b2)


# speedup vs baseline: 3.3165x; 1.0789x over previous
import functools

import jax
import jax.numpy as jnp
from jax import lax
from jax.experimental import pallas as pl
from jax.experimental.pallas import tpu as pltpu
from jax.experimental.pallas import tpu_sc as plsc

NUM_WORDS = 1000000
NUM_LABELS = 100
EMBED_DIM = 32
HIDDEN_DIM = 64
BATCH = 16384

_CW = 65536
_Q = _CW // 4
_SH_CW = _CW.bit_length() - 1
_SH_Q = _Q.bit_length() - 1
_NBLK = pl.cdiv(NUM_WORDS, _CW)
_T2_ROWS = _NBLK * _Q


def _relayout_body(t_ref, o_ref):
    q = _CW // 4
    parts = [t_ref[:, q * a:q * (a + 1)] for a in range(4)]
    v = jnp.concatenate(parts, axis=0)
    o_ref[...] = v.T


def _relayout(tableT):
    return pl.pallas_call(
        _relayout_body,
        grid=(_NBLK,),
        in_specs=[pl.BlockSpec((EMBED_DIM, _CW), lambda i: (0, i))],
        out_specs=pl.BlockSpec((_Q, 128), lambda i: (i, 0)),
        out_shape=jax.ShapeDtypeStruct((_T2_ROWS, 128), jnp.float32),
    )(tableT)


def _make_gather(B: int, D: int):
    info = plsc.get_sparse_core_info()
    NC, NS = info.num_cores, info.num_subcores
    NW = NC * NS
    assert B % (8 * NW) == 0
    b_per_w = B // NW
    mesh = plsc.VectorSubcoreMesh(core_axis_name="c", subcore_axis_name="s")

    @functools.partial(
        pl.kernel,
        mesh=mesh,
        out_type=jax.ShapeDtypeStruct((B, D), jnp.float32),
        scratch_types=[
            pltpu.VMEM((b_per_w,), jnp.int32),
            pltpu.VMEM((b_per_w,), jnp.int32),
            pltpu.VMEM((b_per_w, D), jnp.float32),
            pltpu.SemaphoreType.DMA,
        ],
        compiler_params=pltpu.CompilerParams(use_tc_tiling_on_sc=False),
    )
    def gather_k(idx_hbm, table_hbm, out_hbm, idx_v, idx2_v, rows_v, sem):
        wid = lax.axis_index("s") * NC + lax.axis_index("c")
        base = wid * b_per_w
        pltpu.sync_copy(idx_hbm.at[pl.ds(base, b_per_w)], idx_v)

        def body(j, _):
            w = idx_v[pl.ds(j * 16, 16)]
            i = lax.shift_right_logical(w, _SH_CW)
            q = lax.bitwise_and(w, _CW - 1)
            a = lax.shift_right_logical(q, _SH_Q)
            r = lax.bitwise_and(q, _Q - 1)
            s = lax.bitwise_or(
                lax.shift_left(i * _Q + r, 2), a)
            idx2_v[pl.ds(j * 16, 16)] = s
            return 0

        lax.fori_loop(0, b_per_w // 16, body, 0, unroll=4)

        pltpu.async_copy(table_hbm.at[idx2_v], rows_v, sem).wait()
        pltpu.sync_copy(rows_v, out_hbm.at[pl.ds(base, b_per_w)])

    return gather_k


def _mlp_body(e_ref, w1_ref, b1_ref, w2_ref, b2_ref, o_ref):
    e = e_ref[...]
    ht = lax.dot_general(w1_ref[...], e, (((1,), (1,)), ((), ())),
                         preferred_element_type=jnp.float32)
    ht = jnp.maximum(ht + b1_ref[...], 0.0)
    ot = lax.dot_general(w2_ref[...], ht, (((1,), (0,)), ((), ())),
                         preferred_element_type=jnp.float32)
    o_ref[...] = ot + b2_ref[...]


def _mlp(emb, W1, b1, W2, b2):
    B = emb.shape[0]
    BB = 2048
    grid = (B // BB,)
    out_t = pl.pallas_call(
        _mlp_body,
        grid=grid,
        in_specs=[
            pl.BlockSpec((BB, EMBED_DIM), lambda i: (i, 0)),
            pl.BlockSpec((HIDDEN_DIM, EMBED_DIM), lambda i: (0, 0)),
            pl.BlockSpec((HIDDEN_DIM, 1), lambda i: (0, 0)),
            pl.BlockSpec((NUM_LABELS, HIDDEN_DIM), lambda i: (0, 0)),
            pl.BlockSpec((NUM_LABELS, 1), lambda i: (0, 0)),
        ],
        out_specs=pl.BlockSpec((NUM_LABELS, BB), lambda i: (0, i)),
        out_shape=jax.ShapeDtypeStruct((NUM_LABELS, B), jnp.float32),
    )(emb, W1, b1.reshape(-1, 1), W2, b2.reshape(-1, 1))
    return out_t.T


def kernel(x, table, W1, b1, W2, b2):
    t2 = _relayout(table.T)
    table_lin = t2.reshape(_T2_ROWS * 4, EMBED_DIM)
    gather_k = _make_gather(BATCH, EMBED_DIM)
    emb = gather_k(x.astype(jnp.int32), table_lin)
    return _mlp(emb, W1, b1, W2, b2)


# speedup vs baseline: 3.3901x; 1.0222x over previous
import functools

import jax
import jax.numpy as jnp
from jax import lax
from jax.experimental import pallas as pl
from jax.experimental.pallas import tpu as pltpu
from jax.experimental.pallas import tpu_sc as plsc

NUM_WORDS = 1000000
NUM_LABELS = 100
EMBED_DIM = 32
HIDDEN_DIM = 64
BATCH = 16384

_CW = 65536
_Q = _CW // 4
_SH_CW = _CW.bit_length() - 1
_SH_Q = _Q.bit_length() - 1
_NBLK = pl.cdiv(NUM_WORDS, _CW)
_T2_ROWS = _NBLK * _Q


def _relayout_body(t_ref, o_ref):
    q = _CW // 4
    parts = [t_ref[:, q * a:q * (a + 1)] for a in range(4)]
    v = jnp.concatenate(parts, axis=0)
    o_ref[...] = v.T


def _relayout(tableT):
    return pl.pallas_call(
        _relayout_body,
        grid=(_NBLK,),
        in_specs=[pl.BlockSpec((EMBED_DIM, _CW), lambda i: (0, i))],
        out_specs=pl.BlockSpec((_Q, 128), lambda i: (i, 0)),
        out_shape=jax.ShapeDtypeStruct((_T2_ROWS, 128), jnp.float32),
    )(tableT)


def _make_gather(B: int, D: int):
    info = plsc.get_sparse_core_info()
    NC, NS = info.num_cores, info.num_subcores
    NW = NC * NS
    assert B % (8 * NW) == 0
    b_per_w = B // NW
    mesh = plsc.VectorSubcoreMesh(core_axis_name="c", subcore_axis_name="s")

    @functools.partial(
        pl.kernel,
        mesh=mesh,
        out_type=jax.ShapeDtypeStruct((B, D), jnp.float32),
        scratch_types=[
            pltpu.VMEM((b_per_w,), jnp.int32),
            pltpu.VMEM((b_per_w,), jnp.int32),
            pltpu.VMEM((b_per_w, D), jnp.float32),
            pltpu.SemaphoreType.DMA,
        ],
        compiler_params=pltpu.CompilerParams(use_tc_tiling_on_sc=False),
    )
    def gather_k(idx_hbm, table_hbm, out_hbm, idx_v, idx2_v, rows_v, sem):
        wid = lax.axis_index("s") * NC + lax.axis_index("c")
        base = wid * b_per_w
        pltpu.sync_copy(idx_hbm.at[pl.ds(base, b_per_w)], idx_v)

        def body(j, _):
            w = idx_v[pl.ds(j * 16, 16)]
            i = lax.shift_right_logical(w, _SH_CW)
            q = lax.bitwise_and(w, _CW - 1)
            a = lax.shift_right_logical(q, _SH_Q)
            r = lax.bitwise_and(q, _Q - 1)
            s = lax.bitwise_or(
                lax.shift_left(i * _Q + r, 2), a)
            idx2_v[pl.ds(j * 16, 16)] = s
            return 0

        lax.fori_loop(0, b_per_w // 16, body, 0, unroll=4)

        pltpu.async_copy(table_hbm.at[idx2_v], rows_v, sem).wait()
        pltpu.sync_copy(rows_v, out_hbm.at[pl.ds(base, b_per_w)])

    return gather_k


def _mlp_body(e_ref, w1_ref, b1_ref, w2_ref, b2_ref, o_ref):
    e = e_ref[...]
    ht = lax.dot_general(w1_ref[...], e, (((1,), (1,)), ((), ())),
                         preferred_element_type=jnp.float32)
    ht = jnp.maximum(ht + b1_ref[...], 0.0)
    ot = lax.dot_general(w2_ref[...], ht, (((1,), (0,)), ((), ())),
                         preferred_element_type=jnp.float32)
    o_ref[...] = ot + b2_ref[...]


def _mlp(emb, W1, b1, W2, b2):
    B = emb.shape[0]
    BB = 4096
    grid = (B // BB,)
    out_t = pl.pallas_call(
        _mlp_body,
        grid=grid,
        in_specs=[
            pl.BlockSpec((BB, EMBED_DIM), lambda i: (i, 0)),
            pl.BlockSpec((HIDDEN_DIM, EMBED_DIM), lambda i: (0, 0)),
            pl.BlockSpec((HIDDEN_DIM, 1), lambda i: (0, 0)),
            pl.BlockSpec((NUM_LABELS, HIDDEN_DIM), lambda i: (0, 0)),
            pl.BlockSpec((NUM_LABELS, 1), lambda i: (0, 0)),
        ],
        out_specs=pl.BlockSpec((NUM_LABELS, BB), lambda i: (0, i)),
        out_shape=jax.ShapeDtypeStruct((NUM_LABELS, B), jnp.float32),
    )(emb, W1, b1.reshape(-1, 1), W2, b2.reshape(-1, 1))
    return out_t.T


def kernel(x, table, W1, b1, W2, b2):
    t2 = _relayout(table.T)
    table_lin = t2.reshape(_T2_ROWS * 4, EMBED_DIM)
    gather_k = _make_gather(BATCH, EMBED_DIM)
    emb = gather_k(x.astype(jnp.int32), table_lin)
    return _mlp(emb, W1, b1, W2, b2)


# speedup vs baseline: 3.5887x; 1.0586x over previous
import functools

import jax
import jax.numpy as jnp
from jax import lax
from jax.experimental import pallas as pl
from jax.experimental.pallas import tpu as pltpu
from jax.experimental.pallas import tpu_sc as plsc

NUM_WORDS = 1000000
NUM_LABELS = 100
EMBED_DIM = 32
HIDDEN_DIM = 64
BATCH = 16384

_CW = 65536
_Q = _CW // 4
_SH_CW = _CW.bit_length() - 1
_SH_Q = _Q.bit_length() - 1
_NBLK = pl.cdiv(NUM_WORDS, _CW)
_T2_ROWS = _NBLK * _Q


def _relayout_body(t_ref, o_ref):
    q = _CW // 4
    parts = [t_ref[:, q * a:q * (a + 1)] for a in range(4)]
    v = jnp.concatenate(parts, axis=0)
    o_ref[...] = v.T


def _relayout(tableT):
    return pl.pallas_call(
        _relayout_body,
        grid=(_NBLK,),
        in_specs=[pl.BlockSpec((EMBED_DIM, _CW), lambda i: (0, i))],
        out_specs=pl.BlockSpec((_Q, 128), lambda i: (i, 0)),
        out_shape=jax.ShapeDtypeStruct((_T2_ROWS, 128), jnp.float32),
    )(tableT)


def _make_gather(B: int, D: int):
    info = plsc.get_sparse_core_info()
    NC, NS = info.num_cores, info.num_subcores
    NW = NC * NS
    assert B % (8 * NW) == 0
    b_per_w = B // NW
    mesh = plsc.VectorSubcoreMesh(core_axis_name="c", subcore_axis_name="s")

    @functools.partial(
        pl.kernel,
        mesh=mesh,
        out_type=jax.ShapeDtypeStruct((B, 128), jnp.float32),
        scratch_types=[
            pltpu.VMEM((b_per_w,), jnp.int32),
            pltpu.VMEM((b_per_w,), jnp.int32),
            pltpu.VMEM((b_per_w, D), jnp.float32),
            pltpu.VMEM((b_per_w, 128), jnp.float32),
            pltpu.SemaphoreType.DMA,
        ],
        compiler_params=pltpu.CompilerParams(use_tc_tiling_on_sc=False),
    )
    def gather_k(idx_hbm, table_hbm, out_hbm, idx_v, idx2_v, rows_v, rows128_v, sem):
        wid = lax.axis_index("s") * NC + lax.axis_index("c")
        base = wid * b_per_w
        pltpu.sync_copy(idx_hbm.at[pl.ds(base, b_per_w)], idx_v)

        def body(j, _):
            w = idx_v[pl.ds(j * 16, 16)]
            i = lax.shift_right_logical(w, _SH_CW)
            q = lax.bitwise_and(w, _CW - 1)
            a = lax.shift_right_logical(q, _SH_Q)
            r = lax.bitwise_and(q, _Q - 1)
            s = lax.bitwise_or(
                lax.shift_left(i * _Q + r, 2), a)
            idx2_v[pl.ds(j * 16, 16)] = s
            return 0

        lax.fori_loop(0, b_per_w // 16, body, 0, unroll=4)

        pltpu.async_copy(table_hbm.at[idx2_v], rows_v, sem).wait()
        pltpu.sync_copy(rows_v,
                        out_hbm.at[pl.ds(base, b_per_w), pl.ds(0, D)])

    return gather_k


def _mlp_body(e_ref, w1_ref, b1_ref, w2_ref, b2_ref, o_ref):
    e = e_ref[:, :EMBED_DIM]
    ht = lax.dot_general(w1_ref[...], e, (((1,), (1,)), ((), ())),
                         preferred_element_type=jnp.float32)
    ht = jnp.maximum(ht + b1_ref[...], 0.0)
    ot = lax.dot_general(w2_ref[...], ht, (((1,), (0,)), ((), ())),
                         preferred_element_type=jnp.float32)
    o_ref[...] = ot + b2_ref[...]


def _mlp(emb, W1, b1, W2, b2):
    B = emb.shape[0]
    BB = 4096
    grid = (B // BB,)
    out_t = pl.pallas_call(
        _mlp_body,
        grid=grid,
        in_specs=[
            pl.BlockSpec((BB, 128), lambda i: (i, 0)),
            pl.BlockSpec((HIDDEN_DIM, EMBED_DIM), lambda i: (0, 0)),
            pl.BlockSpec((HIDDEN_DIM, 1), lambda i: (0, 0)),
            pl.BlockSpec((NUM_LABELS, HIDDEN_DIM), lambda i: (0, 0)),
            pl.BlockSpec((NUM_LABELS, 1), lambda i: (0, 0)),
        ],
        out_specs=pl.BlockSpec((NUM_LABELS, BB), lambda i: (0, i)),
        out_shape=jax.ShapeDtypeStruct((NUM_LABELS, B), jnp.float32),
    )(emb, W1, b1.reshape(-1, 1), W2, b2.reshape(-1, 1))
    return out_t.T


def kernel(x, table, W1, b1, W2, b2):
    t2 = _relayout(table.T)
    table_lin = t2.reshape(_T2_ROWS * 4, EMBED_DIM)
    gather_k = _make_gather(BATCH, EMBED_DIM)
    emb = gather_k(x.astype(jnp.int32), table_lin)
    return _mlp(emb, W1, b1, W2, b2)


# speedup vs baseline: 3.6273x; 1.0108x over previous
import functools

import jax
import jax.numpy as jnp
from jax import lax
from jax.experimental import pallas as pl
from jax.experimental.pallas import tpu as pltpu
from jax.experimental.pallas import tpu_sc as plsc

NUM_WORDS = 1000000
NUM_LABELS = 100
EMBED_DIM = 32
HIDDEN_DIM = 64
BATCH = 16384

_CW = 65536
_Q = _CW // 4
_SH_CW = _CW.bit_length() - 1
_SH_Q = _Q.bit_length() - 1
_NBLK = pl.cdiv(NUM_WORDS, _CW)
_T2_ROWS = _NBLK * _Q


def _relayout_body(t_ref, o_ref):
    q = _CW // 4
    parts = [t_ref[:, q * a:q * (a + 1)] for a in range(4)]
    v = jnp.concatenate(parts, axis=0)
    o_ref[...] = v.T


def _relayout(tableT):
    return pl.pallas_call(
        _relayout_body,
        grid=(_NBLK,),
        in_specs=[pl.BlockSpec((EMBED_DIM, _CW), lambda i: (0, i))],
        out_specs=pl.BlockSpec((_Q, 128), lambda i: (i, 0)),
        out_shape=jax.ShapeDtypeStruct((_T2_ROWS, 128), jnp.float32),
    )(tableT)


def _make_gather(B: int, D: int):
    info = plsc.get_sparse_core_info()
    NC, NS = info.num_cores, info.num_subcores
    NW = NC * NS
    assert B % (8 * NW) == 0
    b_per_w = B // NW
    mesh = plsc.VectorSubcoreMesh(core_axis_name="c", subcore_axis_name="s")

    @functools.partial(
        pl.kernel,
        mesh=mesh,
        out_type=jax.ShapeDtypeStruct((B, 128), jnp.float32),
        scratch_types=[
            pltpu.VMEM((b_per_w,), jnp.int32),
            pltpu.VMEM((b_per_w,), jnp.int32),
            pltpu.VMEM((b_per_w, D), jnp.float32),
            pltpu.VMEM((b_per_w, 128), jnp.float32),
            pltpu.SemaphoreType.DMA,
        ],
        compiler_params=pltpu.CompilerParams(use_tc_tiling_on_sc=False),
    )
    def gather_k(idx_hbm, table_hbm, out_hbm, idx_v, idx2_v, rows_v, rows128_v, sem):
        wid = lax.axis_index("s") * NC + lax.axis_index("c")
        base = wid * b_per_w
        pltpu.sync_copy(idx_hbm.at[pl.ds(base, b_per_w)], idx_v)

        def body(j, _):
            w = idx_v[pl.ds(j * 16, 16)]
            i = lax.shift_right_logical(w, _SH_CW)
            q = lax.bitwise_and(w, _CW - 1)
            a = lax.shift_right_logical(q, _SH_Q)
            r = lax.bitwise_and(q, _Q - 1)
            s = lax.bitwise_or(
                lax.shift_left(i * _Q + r, 2), a)
            idx2_v[pl.ds(j * 16, 16)] = s
            return 0

        lax.fori_loop(0, b_per_w // 16, body, 0, unroll=4)

        pltpu.async_copy(table_hbm.at[idx2_v], rows_v, sem).wait()
        pltpu.sync_copy(rows_v,
                        out_hbm.at[pl.ds(base, b_per_w), pl.ds(0, D)])

    return gather_k


def _mlp_body(e_ref, w1_ref, b1_ref, w2_ref, b2_ref, o_ref):
    e = e_ref[:, :EMBED_DIM]
    ht = lax.dot_general(w1_ref[...], e, (((1,), (1,)), ((), ())),
                         preferred_element_type=jnp.float32)
    ht = jnp.maximum(ht + b1_ref[...], 0.0)
    ot = lax.dot_general(w2_ref[...], ht, (((1,), (0,)), ((), ())),
                         preferred_element_type=jnp.float32)
    o_ref[...] = ot + b2_ref[...]


def _mlp(emb, W1, b1, W2, b2):
    B = emb.shape[0]
    BB = 8192
    grid = (B // BB,)
    out_t = pl.pallas_call(
        _mlp_body,
        grid=grid,
        in_specs=[
            pl.BlockSpec((BB, 128), lambda i: (i, 0)),
            pl.BlockSpec((HIDDEN_DIM, EMBED_DIM), lambda i: (0, 0)),
            pl.BlockSpec((HIDDEN_DIM, 1), lambda i: (0, 0)),
            pl.BlockSpec((NUM_LABELS, HIDDEN_DIM), lambda i: (0, 0)),
            pl.BlockSpec((NUM_LABELS, 1), lambda i: (0, 0)),
        ],
        out_specs=pl.BlockSpec((NUM_LABELS, BB), lambda i: (0, i)),
        out_shape=jax.ShapeDtypeStruct((NUM_LABELS, B), jnp.float32),
    )(emb, W1, b1.reshape(-1, 1), W2, b2.reshape(-1, 1))
    return out_t.T


def kernel(x, table, W1, b1, W2, b2):
    t2 = _relayout(table.T)
    table_lin = t2.reshape(_T2_ROWS * 4, EMBED_DIM)
    gather_k = _make_gather(BATCH, EMBED_DIM)
    emb = gather_k(x.astype(jnp.int32), table_lin)
    return _mlp(emb, W1, b1, W2, b2)
